# baseline (speedup 1.0000x reference)
import functools
import math

import jax
import jax.numpy as jnp
from jax import lax
from jax.experimental import pallas as pl
from jax.experimental.pallas import tpu as pltpu
from jax.experimental.pallas import tpu_sc as plsc

NUM_EMB = 100000
INNER = 16
EMB = 96
B = 16384
L = 200


_ROWS_PER_BLOCK = 4000
EMB_PAD = 128


def _transform_body(table_ref, w_ref, b_ref, gamma_ref, beta_ref, out_ref):
    x = jnp.dot(table_ref[...], w_ref[...], preferred_element_type=jnp.float32)
    x = x + b_ref[...]
    mean = jnp.sum(x, axis=-1, keepdims=True) * (1.0 / EMB)
    mask = lax.broadcasted_iota(jnp.int32, x.shape, 1) < EMB
    xc = jnp.where(mask, x - mean, 0.0)
    var = jnp.sum(xc * xc, axis=-1, keepdims=True) * (1.0 / EMB)
    y = xc * lax.rsqrt(var + 1e-5)
    y = y * gamma_ref[...] + beta_ref[...]
    out_ref[...] = y * 0.5 * (1.0 + lax.erf(y * (1.0 / math.sqrt(2.0))))


def _transform_table(table, W, b2, gamma2, beta2):
    grid = (NUM_EMB // _ROWS_PER_BLOCK,)
    return pl.pallas_call(
        _transform_body,
        grid=grid,
        in_specs=[
            pl.BlockSpec((_ROWS_PER_BLOCK, INNER), lambda i: (i, 0)),
            pl.BlockSpec((INNER, EMB_PAD), lambda i: (0, 0)),
            pl.BlockSpec((1, EMB_PAD), lambda i: (0, 0)),
            pl.BlockSpec((1, EMB_PAD), lambda i: (0, 0)),
            pl.BlockSpec((1, EMB_PAD), lambda i: (0, 0)),
        ],
        out_specs=pl.BlockSpec((_ROWS_PER_BLOCK, EMB_PAD), lambda i: (i, 0)),
        out_shape=jax.ShapeDtypeStruct((NUM_EMB, EMB_PAD), jnp.float32),
    )(table, W, b2, gamma2, beta2)


N = B * L
_NC, _NS = 2, 16
_NW = _NC * _NS
_SAMP_W = B // _NW
_CHUNK = 2 * L
_N_CHUNKS = _SAMP_W // 2
_HSZ = (104, 96)
_HOFF = (0, 104)
_LANES = 16
_VPR = EMB // _LANES


@functools.cache
def _make_gather_kernel():
    @functools.partial(
        pl.kernel,
        mesh=plsc.VectorSubcoreMesh(core_axis_name="c", subcore_axis_name="s"),
        out_type=jax.ShapeDtypeStruct((B, L, EMB), jnp.float32),
        scratch_types=[
            pltpu.VMEM((_CHUNK,), jnp.int32),
            pltpu.VMEM((_CHUNK,), jnp.int32),
            pltpu.VMEM((_CHUNK, EMB_PAD), jnp.float32),
            pltpu.VMEM((_CHUNK, EMB_PAD), jnp.float32),
            pltpu.VMEM((L, EMB), jnp.float32),
            pltpu.SemaphoreType.DMA,
            pltpu.SemaphoreType.DMA,
            pltpu.SemaphoreType.DMA,
            pltpu.SemaphoreType.DMA,
        ],
    )
    def _gather_kernel(table_hbm, idx_hbm, out_hbm,
                       idxa, idxb, raw0, raw1, pck0,
                       g0, g1, o0, o1):
        wid = lax.axis_index("s") * _NC + lax.axis_index("c")
        base = wid * _SAMP_W
        tbase = base * L
        raw = (raw0, raw1)
        gsem = (g0, g1)

        def repack(src, r0, dst, nrows):
            def row(r, carry):
                for c in range(_VPR):
                    dst[r, pl.ds(c * _LANES, _LANES)] = (
                        src[r0 + r, pl.ds(c * _LANES, _LANES)])
                return carry
            lax.fori_loop(0, nrows, row, 0, unroll=4)

        idx_v = (idxa, idxb)

        def issue_gather(s, k):
            pltpu.sync_copy(idx_hbm.at[pl.ds(tbase + k * _CHUNK, _CHUNK)],
                            idx_v[s])
            pltpu.async_copy(table_hbm.at[idx_v[s]], raw[s], gsem[s])

        def wait_gather(s, k):
            pltpu.make_async_copy(table_hbm.at[idx_v[s]], raw[s],
                                  gsem[s]).wait()

        issue_gather(0, 0)
        issue_gather(1, 1)

        def handle(s, k):
            wait_gather(s, k)

            for h in range(2):
                j = base + 2 * k + h

                @pl.when(2 * k + h >= 1)
                def _(j=j):
                    pltpu.make_async_copy(pck0, out_hbm.at[j - 1],
                                          o0).wait()

                repack(raw[s], h * L, pck0, L)
                pltpu.async_copy(pck0, out_hbm.at[j], o0)

            @pl.when(k + 2 < _N_CHUNKS)
            def _():
                issue_gather(s, k + 2)

        def body(i2, carry):
            handle(0, 2 * i2)
            handle(1, 2 * i2 + 1)
            return carry

        lax.fori_loop(0, _N_CHUNKS // 2, body, 0)

        pltpu.make_async_copy(pck0, out_hbm.at[base + _SAMP_W - 1],
                              o0).wait()

    return _gather_kernel


def kernel(embryo_type, table, W, b, gamma, beta):
    pad = EMB_PAD - EMB
    table2 = _transform_table(
        table,
        jnp.pad(W, ((0, 0), (0, pad))),
        jnp.pad(b.reshape(1, EMB), ((0, 0), (0, pad))),
        jnp.pad(gamma.reshape(1, EMB), ((0, 0), (0, pad))),
        jnp.pad(beta.reshape(1, EMB), ((0, 0), (0, pad))),
    )
    idx = embryo_type.reshape(N).astype(jnp.int32)
    return _make_gather_kernel()(table2, idx)


# speedup vs baseline: 1.3780x; 1.3780x over previous
import functools
import math

import jax
import jax.numpy as jnp
from jax import lax
from jax.experimental import pallas as pl
from jax.experimental.pallas import tpu as pltpu
from jax.experimental.pallas import tpu_sc as plsc

NUM_EMB = 100000
INNER = 16
EMB = 96
B = 16384
L = 200


_ROWS_PER_BLOCK = 4000
EMB_PAD = 128


def _transform_body(table_ref, w_ref, b_ref, gamma_ref, beta_ref, out_ref):
    x = jnp.dot(table_ref[...], w_ref[...], preferred_element_type=jnp.float32)
    x = x + b_ref[...]
    mean = jnp.sum(x, axis=-1, keepdims=True) * (1.0 / EMB)
    mask = lax.broadcasted_iota(jnp.int32, x.shape, 1) < EMB
    xc = jnp.where(mask, x - mean, 0.0)
    var = jnp.sum(xc * xc, axis=-1, keepdims=True) * (1.0 / EMB)
    y = xc * lax.rsqrt(var + 1e-5)
    y = y * gamma_ref[...] + beta_ref[...]
    out_ref[...] = y * 0.5 * (1.0 + lax.erf(y * (1.0 / math.sqrt(2.0))))


def _transform_table(table, W, b2, gamma2, beta2):
    grid = (NUM_EMB // _ROWS_PER_BLOCK,)
    return pl.pallas_call(
        _transform_body,
        grid=grid,
        in_specs=[
            pl.BlockSpec((_ROWS_PER_BLOCK, INNER), lambda i: (i, 0)),
            pl.BlockSpec((INNER, EMB_PAD), lambda i: (0, 0)),
            pl.BlockSpec((1, EMB_PAD), lambda i: (0, 0)),
            pl.BlockSpec((1, EMB_PAD), lambda i: (0, 0)),
            pl.BlockSpec((1, EMB_PAD), lambda i: (0, 0)),
        ],
        out_specs=pl.BlockSpec((_ROWS_PER_BLOCK, EMB_PAD), lambda i: (i, 0)),
        out_shape=jax.ShapeDtypeStruct((NUM_EMB, EMB_PAD), jnp.float32),
    )(table, W, b2, gamma2, beta2)


N = B * L
_NC, _NS = 2, 16
_NW = _NC * _NS
_PER_W = N // _NW
_CHUNK = 400
_N_CHUNKS = _PER_W // _CHUNK


@functools.cache
def _make_gather_kernel():
    @functools.partial(
        pl.kernel,
        mesh=plsc.VectorSubcoreMesh(core_axis_name="c", subcore_axis_name="s"),
        out_type=jax.ShapeDtypeStruct((N, EMB_PAD), jnp.float32),
        scratch_types=[
            pltpu.VMEM((_CHUNK,), jnp.int32),
            pltpu.VMEM((_CHUNK,), jnp.int32),
            pltpu.VMEM((_CHUNK, EMB_PAD), jnp.float32),
            pltpu.VMEM((_CHUNK, EMB_PAD), jnp.float32),
            pltpu.SemaphoreType.DMA,
            pltpu.SemaphoreType.DMA,
            pltpu.SemaphoreType.DMA,
            pltpu.SemaphoreType.DMA,
        ],
    )
    def _gather_kernel(table_hbm, idx_hbm, out_hbm,
                       idxa, idxb, raw0, raw1,
                       g0, g1, o0, o1):
        wid = lax.axis_index("s") * _NC + lax.axis_index("c")
        tbase = wid * _PER_W
        raw = (raw0, raw1)
        gsem = (g0, g1)
        osem = (o0, o1)
        idx_v = (idxa, idxb)

        def issue_gather(s, k):
            pltpu.sync_copy(idx_hbm.at[pl.ds(tbase + k * _CHUNK, _CHUNK)],
                            idx_v[s])
            pltpu.async_copy(table_hbm.at[idx_v[s]], raw[s], gsem[s])

        def wait_gather(s, k):
            pltpu.make_async_copy(table_hbm.at[idx_v[s]], raw[s],
                                  gsem[s]).wait()

        def out_dst(k):
            return out_hbm.at[pl.ds(tbase + k * _CHUNK, _CHUNK)]

        issue_gather(0, 0)
        issue_gather(1, 1)

        def handle(s, k):
            wait_gather(s, k)
            pltpu.async_copy(raw[s], out_dst(k), osem[s])

            @pl.when(k + 2 < _N_CHUNKS)
            def _():
                pltpu.make_async_copy(raw[s], out_dst(k), osem[s]).wait()
                issue_gather(s, k + 2)

        def body(i2, carry):
            handle(0, 2 * i2)
            handle(1, 2 * i2 + 1)
            return carry

        lax.fori_loop(0, _N_CHUNKS // 2, body, 0)

        pltpu.make_async_copy(raw[0], out_dst(_N_CHUNKS - 2),
                              osem[0]).wait()
        pltpu.make_async_copy(raw[1], out_dst(_N_CHUNKS - 1),
                              osem[1]).wait()

    return _gather_kernel


def kernel(embryo_type, table, W, b, gamma, beta):
    pad = EMB_PAD - EMB
    table2 = _transform_table(
        table,
        jnp.pad(W, ((0, 0), (0, pad))),
        jnp.pad(b.reshape(1, EMB), ((0, 0), (0, pad))),
        jnp.pad(gamma.reshape(1, EMB), ((0, 0), (0, pad))),
        jnp.pad(beta.reshape(1, EMB), ((0, 0), (0, pad))),
    )
    idx = embryo_type.reshape(N).astype(jnp.int32)
    out = _make_gather_kernel()(table2, idx)
    return out[:, :EMB].reshape(B, L, EMB)
